# baseline (speedup 1.0000x reference)
import dataclasses
import functools

import jax
import jax.numpy as jnp
from jax.experimental import pallas as pl
from jax.experimental.pallas import tpu as pltpu
from jax.experimental.pallas import tpu_sc as plsc

EPS = 1e-5
TB = 512


def _sc_compiler_params():
    cp = pltpu.CompilerParams(use_tc_tiling_on_sc=True)
    if "needs_layout_passes" in pltpu.CompilerParams.__dataclass_fields__:
        cp = dataclasses.replace(cp, needs_layout_passes=False)
    return cp


def _sc_repack(embT, tail_slabs):
    F, D, V = embT.shape
    C = 2048
    mesh = plsc.VectorSubcoreMesh(core_axis_name="core", subcore_axis_name="subcore")
    info = plsc.get_sparse_core_info()
    NW = info.num_cores * info.num_subcores
    L = info.num_lanes

    NBIG = V // C
    VT = NBIG * C
    C2 = ((V - VT) // 128) * 128
    VT2 = VT + C2
    TROWS = (V - VT2) * D // 128

    @functools.partial(
        pl.kernel,
        out_type=jax.ShapeDtypeStruct((F, V // 8, 8 * D), jnp.float32),
        mesh=mesh,
        scratch_types=[
            pltpu.VMEM((D, C), jnp.float32),
            pltpu.VMEM((C * D // 128, 128), jnp.float32),
            pltpu.SemaphoreType.DMA,
        ],
        compiler_params=_sc_compiler_params(),
    )
    def k(embT_hbm, tail_hbm, out_hbm, buf_v, tbuf_v, sem):
        wid = (jax.lax.axis_index("subcore") * info.num_cores
               + jax.lax.axis_index("core"))

        def transpose_rows(nv):
            @pl.loop(0, nv, step=8)
            def _(v):
                lanes = jax.lax.iota(jnp.int32, L)
                for vv in range(8):
                    vi = v + vv
                    vals = plsc.load_gather(
                        buf_v, [lanes, jnp.zeros((L,), jnp.int32) + vi])
                    flat = vi * D + lanes
                    plsc.store_scatter(
                        tbuf_v,
                        [jax.lax.shift_right_logical(flat, 7), flat & 127],
                        vals)

        @pl.loop(wid, F * NBIG, step=NW)
        def _(u):
            f = u // NBIG
            v0 = pl.multiple_of((u % NBIG) * C, C)
            pltpu.sync_copy(embT_hbm.at[f, :, pl.ds(v0, C)], buf_v)
            transpose_rows(C)
            r0 = pl.multiple_of(v0 * D // 128, C * D // 128)
            pltpu.sync_copy(tbuf_v, out_hbm.at[f, pl.ds(r0, C * D // 128)])

        if C2 > 0:
            @pl.loop(wid, F, step=NW)
            def _(f):
                pltpu.sync_copy(embT_hbm.at[f, :, pl.ds(VT, C2)],
                                buf_v.at[:, pl.ds(0, C2)])
                transpose_rows(C2)
                pltpu.sync_copy(tbuf_v.at[pl.ds(0, C2 * D // 128)],
                                out_hbm.at[f, pl.ds(VT * D // 128, C2 * D // 128)])

        if V > VT2:
            @pl.loop(wid, F, step=NW)
            def _(f):
                pltpu.sync_copy(tail_hbm.at[f],
                                out_hbm.at[f, pl.ds(VT2 * D // 128, TROWS)])

    return k(embT, tail_slabs)


def _sc_gather(emb8, fm_flat, xcT, D):
    F, nslab, slab_w = emb8.shape
    per_slab = slab_w // D
    V = nslab * per_slab
    B = xcT.shape[1]
    mesh = plsc.VectorSubcoreMesh(core_axis_name="core", subcore_axis_name="subcore")
    info = plsc.get_sparse_core_info()
    NW = info.num_cores * info.num_subcores
    S = B // NW
    L = info.num_lanes

    @functools.partial(
        pl.kernel,
        out_type=[
            jax.ShapeDtypeStruct((F, D, B), jnp.float32),
            jax.ShapeDtypeStruct((B,), jnp.float32),
        ],
        mesh=mesh,
        scratch_types=[
            pltpu.VMEM((S,), jnp.int32),
            pltpu.VMEM((S,), jnp.int32),
            pltpu.VMEM((S, slab_w), jnp.float32),
            pltpu.VMEM((D, S), jnp.float32),
            pltpu.VMEM((S,), jnp.float32),
            pltpu.VMEM((S,), jnp.float32),
            pltpu.SemaphoreType.DMA,
        ],
        compiler_params=_sc_compiler_params(),
    )
    def k(emb_hbm, fm_hbm, xc_hbm, oemb_hbm, ofm_hbm,
          idx_v, sidx_v, slab_v, rowsT_v, fmv_v, fmacc_v, sem):
        wid = (jax.lax.axis_index("subcore") * info.num_cores
               + jax.lax.axis_index("core"))
        base = wid * S

        @pl.loop(0, S, step=L)
        def _(j):
            fmacc_v[pl.ds(j, L)] = jnp.zeros((L,), jnp.float32)

        @pl.loop(0, F)
        def _(f):
            pltpu.sync_copy(xc_hbm.at[f, pl.ds(base, S)], idx_v)

            @pl.loop(0, S, step=L)
            def _(j):
                sidx_v[pl.ds(j, L)] = jax.lax.shift_right_logical(
                    idx_v[pl.ds(j, L)], 3)

            pltpu.async_copy(emb_hbm.at[f].at[sidx_v], slab_v, sem).wait()

            @pl.loop(0, S, step=L)
            def _(i):
                lanes = jax.lax.iota(jnp.int32, L)
                rows = lanes + i
                colb = (idx_v[pl.ds(i, L)] & (per_slab - 1)) * D
                for d in range(D):
                    vals = plsc.load_gather(slab_v, [rows, colb + d])
                    rowsT_v[d, pl.ds(i, L)] = vals

            pltpu.sync_copy(rowsT_v, oemb_hbm.at[f, :, pl.ds(base, S)])

            off = f * V

            @pl.loop(0, S, step=L)
            def _(j):
                sidx_v[pl.ds(j, L)] = idx_v[pl.ds(j, L)] + off

            pltpu.async_copy(fm_hbm.at[sidx_v], fmv_v, sem).wait()

            @pl.loop(0, S, step=L)
            def _(j):
                fmacc_v[pl.ds(j, L)] = fmacc_v[pl.ds(j, L)] + fmv_v[pl.ds(j, L)]

        pltpu.sync_copy(fmacc_v, ofm_hbm.at[pl.ds(base, S)])

    return k(emb8, fm_flat, xcT)


def _tc1_body(emb_ref, xn_ref, fm1_ref, w1a_ref, w1b_ref, b1_ref, b3_ref,
              h1_ref, fmsum_ref, s_ref, ss_ref, *, nf):
    h = jax.lax.dot(w1b_ref[...], xn_ref[...],
                    precision=jax.lax.Precision.HIGHEST)
    h = h + b1_ref[...]
    s16 = None
    sq = None
    for f in range(nf):
        e = emb_ref[f]
        s16 = e if s16 is None else s16 + e
        esq = jnp.sum(e * e, axis=0)
        sq = esq if sq is None else sq + esq
        d = e.shape[0]
        h = h + jnp.dot(w1a_ref[:, pl.ds(f * d, d)], e.astype(jnp.bfloat16),
                        preferred_element_type=jnp.float32)
    fm2 = 0.5 * (jnp.sum(s16 * s16, axis=0) - sq)
    fmsum_ref[...] = (fm1_ref[0, :] + fm2 + b3_ref[0, 0])[None, :]
    h1_ref[...] = h

    @pl.when(pl.program_id(0) == 0)
    def _():
        s_ref[...] = jnp.zeros_like(s_ref)
        ss_ref[...] = jnp.zeros_like(ss_ref)

    s_ref[...] += jnp.sum(h, axis=1, keepdims=True)
    ss_ref[...] += jnp.sum(h * h, axis=1, keepdims=True)


def _tc2_body(h1_ref, s_ref, ss_ref, g1_ref, be1_ref, w2_ref, b2_ref,
              h2_ref, s2_ref, ss2_ref, *, batch):
    mean = s_ref[...] * (1.0 / batch)
    var = ss_ref[...] * (1.0 / batch) - mean * mean
    inv = g1_ref[...] / jnp.sqrt(var + EPS)
    a = jnp.maximum(h1_ref[...] * inv + (be1_ref[...] - mean * inv), 0.0)
    h = jnp.dot(w2_ref[...], a.astype(jnp.bfloat16),
                preferred_element_type=jnp.float32)
    h = h + b2_ref[...]
    h2_ref[...] = h

    @pl.when(pl.program_id(0) == 0)
    def _():
        s2_ref[...] = jnp.zeros_like(s2_ref)
        ss2_ref[...] = jnp.zeros_like(ss2_ref)

    s2_ref[...] += jnp.sum(h, axis=1, keepdims=True)
    ss2_ref[...] += jnp.sum(h * h, axis=1, keepdims=True)


def _tc3_body(h2_ref, s2_ref, ss2_ref, g2_ref, be2_ref, w3_ref, fmsum_ref,
              out_ref, *, batch):
    mean = s2_ref[...] * (1.0 / batch)
    var = ss2_ref[...] * (1.0 / batch) - mean * mean
    inv = g2_ref[...] / jnp.sqrt(var + EPS)
    a = jnp.maximum(h2_ref[...] * inv + (be2_ref[...] - mean * inv), 0.0)
    dnn = jnp.sum(a * w3_ref[...], axis=0)
    logit = dnn + fmsum_ref[0, :]
    out_ref[...] = jax.nn.sigmoid(logit)[None, :]


def kernel(x_cat, x_num, emb_tables, fm_table, offsets,
           W1, b1, g1, be1, W2, b2, g2, be2, W3, b3):
    B, F = x_cat.shape
    _, V, D = emb_tables.shape
    NUM = x_num.shape[1]
    H = W1.shape[1]
    NB = B // TB

    xcT = jnp.transpose(x_cat).astype(jnp.int32)
    embT = jnp.transpose(emb_tables, (0, 2, 1))

    cutoff = (V // 128) * 128
    tail_slabs = jnp.transpose(embT[:, :, cutoff:], (0, 2, 1)).reshape(
        F, (V - cutoff) * D // (8 * D), 8 * D)
    emb8 = _sc_repack(embT, tail_slabs)
    emb3, fm_first = _sc_gather(emb8, fm_table.reshape(F * V), xcT, D)
    fm1r = fm_first.reshape(1, B)

    xnT = jnp.transpose(x_num)
    w1aT = jnp.transpose(W1[:F * D]).astype(jnp.bfloat16)
    w1bT = jnp.transpose(W1[F * D:])
    w2T = jnp.transpose(W2).astype(jnp.bfloat16)
    b1c = b1.reshape(H, 1)
    b2c = b2.reshape(H, 1)
    g1c = g1.reshape(H, 1)
    be1c = be1.reshape(H, 1)
    g2c = g2.reshape(H, 1)
    be2c = be2.reshape(H, 1)
    w3c = W3.reshape(H, 1)
    b3r = b3.reshape(1, 1)

    const = lambda shape: pl.BlockSpec(shape, lambda i: (0, 0))
    col = lambda shape: pl.BlockSpec(shape, lambda i: (0, i))

    f32 = jnp.float32
    h1, fmsum, s1, ss1 = pl.pallas_call(
        functools.partial(_tc1_body, nf=F),
        grid=(NB,),
        in_specs=[
            pl.BlockSpec((F, D, TB), lambda i: (0, 0, i)),
            col((NUM, TB)), col((1, TB)),
            const((H, F * D)), const((H, NUM)),
            const((H, 1)), const((1, 1)),
        ],
        out_specs=[col((H, TB)), col((1, TB)), const((H, 1)), const((H, 1))],
        out_shape=[
            jax.ShapeDtypeStruct((H, B), f32),
            jax.ShapeDtypeStruct((1, B), f32),
            jax.ShapeDtypeStruct((H, 1), f32),
            jax.ShapeDtypeStruct((H, 1), f32),
        ],
    )(emb3, xnT, fm1r, w1aT, w1bT, b1c, b3r)

    h2, s2, ss2 = pl.pallas_call(
        functools.partial(_tc2_body, batch=B),
        grid=(NB,),
        in_specs=[
            col((H, TB)), const((H, 1)), const((H, 1)),
            const((H, 1)), const((H, 1)), const((H, H)), const((H, 1)),
        ],
        out_specs=[col((H, TB)), const((H, 1)), const((H, 1))],
        out_shape=[
            jax.ShapeDtypeStruct((H, B), f32),
            jax.ShapeDtypeStruct((H, 1), f32),
            jax.ShapeDtypeStruct((H, 1), f32),
        ],
    )(h1, s1, ss1, g1c, be1c, w2T, b2c)

    out2d = pl.pallas_call(
        functools.partial(_tc3_body, batch=B),
        grid=(NB,),
        in_specs=[
            col((H, TB)), const((H, 1)), const((H, 1)),
            const((H, 1)), const((H, 1)), const((H, 1)), col((1, TB)),
        ],
        out_specs=col((1, TB)),
        out_shape=jax.ShapeDtypeStruct((1, B), f32),
    )(h2, s2, ss2, g2c, be2c, w3c, fmsum)

    return out2d.reshape(B)


# speedup vs baseline: 1.7578x; 1.7578x over previous
import dataclasses
import functools

import jax
import jax.numpy as jnp
from jax.experimental import pallas as pl
from jax.experimental.pallas import tpu as pltpu
from jax.experimental.pallas import tpu_sc as plsc

EPS = 1e-5
TB = 512


def _sc_compiler_params():
    cp = pltpu.CompilerParams(use_tc_tiling_on_sc=True)
    if "needs_layout_passes" in pltpu.CompilerParams.__dataclass_fields__:
        cp = dataclasses.replace(cp, needs_layout_passes=False)
    return cp


def _sc_repack(embT, tail_slabs):
    F, D, V = embT.shape
    C = 2048
    mesh = plsc.VectorSubcoreMesh(core_axis_name="core", subcore_axis_name="subcore")
    info = plsc.get_sparse_core_info()
    NW = info.num_cores * info.num_subcores
    L = info.num_lanes

    NBIG = V // C
    VT = NBIG * C
    C2 = ((V - VT) // 128) * 128
    VT2 = VT + C2
    TROWS = (V - VT2) * D // 128

    @functools.partial(
        pl.kernel,
        out_type=jax.ShapeDtypeStruct((F, V // 8, 8 * D), jnp.float32),
        mesh=mesh,
        scratch_types=[
            pltpu.VMEM((D, C), jnp.float32),
            pltpu.VMEM((C * D // 128, 128), jnp.float32),
            pltpu.SemaphoreType.DMA,
        ],
        compiler_params=_sc_compiler_params(),
    )
    def k(embT_hbm, tail_hbm, out_hbm, buf_v, tbuf_v, sem):
        wid = (jax.lax.axis_index("subcore") * info.num_cores
               + jax.lax.axis_index("core"))

        def transpose_rows(nv):
            @pl.loop(0, nv, step=L)
            def _(v):
                vv = jax.lax.iota(jnp.int32, L) + v
                rowv = jax.lax.shift_right_logical(vv, 3)
                colb = (vv & 7) * D
                for d in range(D):
                    vals = buf_v[d, pl.ds(v, L)]
                    plsc.store_scatter(tbuf_v, [rowv, colb + d], vals)

        @pl.loop(wid, F * NBIG, step=NW)
        def _(u):
            f = u // NBIG
            v0 = pl.multiple_of((u % NBIG) * C, C)
            pltpu.sync_copy(embT_hbm.at[f, :, pl.ds(v0, C)], buf_v)
            transpose_rows(C)
            r0 = pl.multiple_of(v0 * D // 128, C * D // 128)
            pltpu.sync_copy(tbuf_v, out_hbm.at[f, pl.ds(r0, C * D // 128)])

        if C2 > 0:
            @pl.loop(wid, F, step=NW)
            def _(f):
                pltpu.sync_copy(embT_hbm.at[f, :, pl.ds(VT, C2)],
                                buf_v.at[:, pl.ds(0, C2)])
                transpose_rows(C2)
                pltpu.sync_copy(tbuf_v.at[pl.ds(0, C2 * D // 128)],
                                out_hbm.at[f, pl.ds(VT * D // 128, C2 * D // 128)])

        if V > VT2:
            @pl.loop(wid, F, step=NW)
            def _(f):
                pltpu.sync_copy(tail_hbm.at[f],
                                out_hbm.at[f, pl.ds(VT2 * D // 128, TROWS)])

    return k(embT, tail_slabs)


def _sc_gather(emb8, fm_flat, xcT, D):
    F, nslab, slab_w = emb8.shape
    per_slab = slab_w // D
    V = nslab * per_slab
    B = xcT.shape[1]
    mesh = plsc.VectorSubcoreMesh(core_axis_name="core", subcore_axis_name="subcore")
    info = plsc.get_sparse_core_info()
    NW = info.num_cores * info.num_subcores
    S = B // NW
    L = info.num_lanes

    @functools.partial(
        pl.kernel,
        out_type=[
            jax.ShapeDtypeStruct((F, D, B), jnp.float32),
            jax.ShapeDtypeStruct((B,), jnp.float32),
        ],
        mesh=mesh,
        scratch_types=[
            pltpu.VMEM((S,), jnp.int32),
            pltpu.VMEM((S,), jnp.int32),
            pltpu.VMEM((S, slab_w), jnp.float32),
            pltpu.VMEM((D, S), jnp.float32),
            pltpu.VMEM((S,), jnp.float32),
            pltpu.VMEM((S,), jnp.float32),
            pltpu.SemaphoreType.DMA,
        ],
        compiler_params=_sc_compiler_params(),
    )
    def k(emb_hbm, fm_hbm, xc_hbm, oemb_hbm, ofm_hbm,
          idx_v, sidx_v, slab_v, rowsT_v, fmv_v, fmacc_v, sem):
        wid = (jax.lax.axis_index("subcore") * info.num_cores
               + jax.lax.axis_index("core"))
        base = wid * S

        @pl.loop(0, S, step=L)
        def _(j):
            fmacc_v[pl.ds(j, L)] = jnp.zeros((L,), jnp.float32)

        @pl.loop(0, F)
        def _(f):
            pltpu.sync_copy(xc_hbm.at[f, pl.ds(base, S)], idx_v)

            @pl.loop(0, S, step=L)
            def _(j):
                sidx_v[pl.ds(j, L)] = jax.lax.shift_right_logical(
                    idx_v[pl.ds(j, L)], 3)

            pltpu.async_copy(emb_hbm.at[f].at[sidx_v], slab_v, sem).wait()

            @pl.loop(0, S, step=L)
            def _(i):
                lanes = jax.lax.iota(jnp.int32, L)
                rows = lanes + i
                colb = (idx_v[pl.ds(i, L)] & (per_slab - 1)) * D
                for d in range(D):
                    vals = plsc.load_gather(slab_v, [rows, colb + d])
                    rowsT_v[d, pl.ds(i, L)] = vals

            pltpu.sync_copy(rowsT_v, oemb_hbm.at[f, :, pl.ds(base, S)])

            off = f * V

            @pl.loop(0, S, step=L)
            def _(j):
                sidx_v[pl.ds(j, L)] = idx_v[pl.ds(j, L)] + off

            pltpu.async_copy(fm_hbm.at[sidx_v], fmv_v, sem).wait()

            @pl.loop(0, S, step=L)
            def _(j):
                fmacc_v[pl.ds(j, L)] = fmacc_v[pl.ds(j, L)] + fmv_v[pl.ds(j, L)]

        pltpu.sync_copy(fmacc_v, ofm_hbm.at[pl.ds(base, S)])

    return k(emb8, fm_flat, xcT)


def _tc1_body(emb_ref, xn_ref, fm1_ref, w1a_ref, w1b_ref, b1_ref, b3_ref,
              h1_ref, fmsum_ref, s_ref, ss_ref, *, nf):
    h = jax.lax.dot(w1b_ref[...], xn_ref[...],
                    precision=jax.lax.Precision.HIGHEST)
    h = h + b1_ref[...]
    s16 = None
    sq = None
    for f in range(nf):
        e = emb_ref[f]
        s16 = e if s16 is None else s16 + e
        esq = jnp.sum(e * e, axis=0)
        sq = esq if sq is None else sq + esq
        d = e.shape[0]
        h = h + jnp.dot(w1a_ref[:, pl.ds(f * d, d)], e.astype(jnp.bfloat16),
                        preferred_element_type=jnp.float32)
    fm2 = 0.5 * (jnp.sum(s16 * s16, axis=0) - sq)
    fmsum_ref[...] = (fm1_ref[0, :] + fm2 + b3_ref[0, 0])[None, :]
    h1_ref[...] = h

    @pl.when(pl.program_id(0) == 0)
    def _():
        s_ref[...] = jnp.zeros_like(s_ref)
        ss_ref[...] = jnp.zeros_like(ss_ref)

    s_ref[...] += jnp.sum(h, axis=1, keepdims=True)
    ss_ref[...] += jnp.sum(h * h, axis=1, keepdims=True)


def _tc2_body(h1_ref, s_ref, ss_ref, g1_ref, be1_ref, w2_ref, b2_ref,
              h2_ref, s2_ref, ss2_ref, *, batch):
    mean = s_ref[...] * (1.0 / batch)
    var = ss_ref[...] * (1.0 / batch) - mean * mean
    inv = g1_ref[...] / jnp.sqrt(var + EPS)
    a = jnp.maximum(h1_ref[...] * inv + (be1_ref[...] - mean * inv), 0.0)
    h = jnp.dot(w2_ref[...], a.astype(jnp.bfloat16),
                preferred_element_type=jnp.float32)
    h = h + b2_ref[...]
    h2_ref[...] = h

    @pl.when(pl.program_id(0) == 0)
    def _():
        s2_ref[...] = jnp.zeros_like(s2_ref)
        ss2_ref[...] = jnp.zeros_like(ss2_ref)

    s2_ref[...] += jnp.sum(h, axis=1, keepdims=True)
    ss2_ref[...] += jnp.sum(h * h, axis=1, keepdims=True)


def _tc3_body(h2_ref, s2_ref, ss2_ref, g2_ref, be2_ref, w3_ref, fmsum_ref,
              out_ref, *, batch):
    mean = s2_ref[...] * (1.0 / batch)
    var = ss2_ref[...] * (1.0 / batch) - mean * mean
    inv = g2_ref[...] / jnp.sqrt(var + EPS)
    a = jnp.maximum(h2_ref[...] * inv + (be2_ref[...] - mean * inv), 0.0)
    dnn = jnp.sum(a * w3_ref[...], axis=0)
    logit = dnn + fmsum_ref[0, :]
    out_ref[...] = jax.nn.sigmoid(logit)[None, :]


def kernel(x_cat, x_num, emb_tables, fm_table, offsets,
           W1, b1, g1, be1, W2, b2, g2, be2, W3, b3):
    B, F = x_cat.shape
    _, V, D = emb_tables.shape
    NUM = x_num.shape[1]
    H = W1.shape[1]
    NB = B // TB

    xcT = jnp.transpose(x_cat).astype(jnp.int32)
    embT = jnp.transpose(emb_tables, (0, 2, 1))

    cutoff = (V // 128) * 128
    tail_slabs = jnp.transpose(embT[:, :, cutoff:], (0, 2, 1)).reshape(
        F, (V - cutoff) * D // (8 * D), 8 * D)
    emb8 = _sc_repack(embT, tail_slabs)
    emb3, fm_first = _sc_gather(emb8, fm_table.reshape(F * V), xcT, D)
    fm1r = fm_first.reshape(1, B)

    xnT = jnp.transpose(x_num)
    w1aT = jnp.transpose(W1[:F * D]).astype(jnp.bfloat16)
    w1bT = jnp.transpose(W1[F * D:])
    w2T = jnp.transpose(W2).astype(jnp.bfloat16)
    b1c = b1.reshape(H, 1)
    b2c = b2.reshape(H, 1)
    g1c = g1.reshape(H, 1)
    be1c = be1.reshape(H, 1)
    g2c = g2.reshape(H, 1)
    be2c = be2.reshape(H, 1)
    w3c = W3.reshape(H, 1)
    b3r = b3.reshape(1, 1)

    const = lambda shape: pl.BlockSpec(shape, lambda i: (0, 0))
    col = lambda shape: pl.BlockSpec(shape, lambda i: (0, i))

    f32 = jnp.float32
    h1, fmsum, s1, ss1 = pl.pallas_call(
        functools.partial(_tc1_body, nf=F),
        grid=(NB,),
        in_specs=[
            pl.BlockSpec((F, D, TB), lambda i: (0, 0, i)),
            col((NUM, TB)), col((1, TB)),
            const((H, F * D)), const((H, NUM)),
            const((H, 1)), const((1, 1)),
        ],
        out_specs=[col((H, TB)), col((1, TB)), const((H, 1)), const((H, 1))],
        out_shape=[
            jax.ShapeDtypeStruct((H, B), f32),
            jax.ShapeDtypeStruct((1, B), f32),
            jax.ShapeDtypeStruct((H, 1), f32),
            jax.ShapeDtypeStruct((H, 1), f32),
        ],
    )(emb3, xnT, fm1r, w1aT, w1bT, b1c, b3r)

    h2, s2, ss2 = pl.pallas_call(
        functools.partial(_tc2_body, batch=B),
        grid=(NB,),
        in_specs=[
            col((H, TB)), const((H, 1)), const((H, 1)),
            const((H, 1)), const((H, 1)), const((H, H)), const((H, 1)),
        ],
        out_specs=[col((H, TB)), const((H, 1)), const((H, 1))],
        out_shape=[
            jax.ShapeDtypeStruct((H, B), f32),
            jax.ShapeDtypeStruct((H, 1), f32),
            jax.ShapeDtypeStruct((H, 1), f32),
        ],
    )(h1, s1, ss1, g1c, be1c, w2T, b2c)

    out2d = pl.pallas_call(
        functools.partial(_tc3_body, batch=B),
        grid=(NB,),
        in_specs=[
            col((H, TB)), const((H, 1)), const((H, 1)),
            const((H, 1)), const((H, 1)), const((H, 1)), col((1, TB)),
        ],
        out_specs=col((1, TB)),
        out_shape=jax.ShapeDtypeStruct((1, B), f32),
    )(h2, s2, ss2, g2c, be2c, w3c, fmsum)

    return out2d.reshape(B)


# speedup vs baseline: 1.7721x; 1.0081x over previous
import dataclasses
import functools

import jax
import jax.numpy as jnp
from jax.experimental import pallas as pl
from jax.experimental.pallas import tpu as pltpu
from jax.experimental.pallas import tpu_sc as plsc

EPS = 1e-5
TB = 512


def _sc_compiler_params():
    cp = pltpu.CompilerParams(use_tc_tiling_on_sc=True)
    if "needs_layout_passes" in pltpu.CompilerParams.__dataclass_fields__:
        cp = dataclasses.replace(cp, needs_layout_passes=False)
    return cp


def _sc_repack(embT, tail_slabs):
    F, D, V = embT.shape
    C = 3968
    mesh = plsc.VectorSubcoreMesh(core_axis_name="core", subcore_axis_name="subcore")
    info = plsc.get_sparse_core_info()
    NW = info.num_cores * info.num_subcores
    L = info.num_lanes

    NBIG = V // C
    VT = NBIG * C
    C2 = ((V - VT) // 128) * 128
    VT2 = VT + C2
    TROWS = (V - VT2) * D // 128

    @functools.partial(
        pl.kernel,
        out_type=jax.ShapeDtypeStruct((F, V // 8, 8 * D), jnp.float32),
        mesh=mesh,
        scratch_types=[
            pltpu.VMEM((D, C), jnp.float32),
            pltpu.VMEM((C * D // 128, 128), jnp.float32),
            pltpu.SemaphoreType.DMA,
        ],
        compiler_params=_sc_compiler_params(),
    )
    def k(embT_hbm, tail_hbm, out_hbm, buf_v, tbuf_v, sem):
        wid = (jax.lax.axis_index("subcore") * info.num_cores
               + jax.lax.axis_index("core"))

        def transpose_rows(nv):
            @pl.loop(0, nv, step=L)
            def _(v):
                vv = jax.lax.iota(jnp.int32, L) + v
                rowv = jax.lax.shift_right_logical(vv, 3)
                colb = (vv & 7) * D
                for d in range(D):
                    vals = buf_v[d, pl.ds(v, L)]
                    plsc.store_scatter(tbuf_v, [rowv, colb + d], vals)

        @pl.loop(wid, F * NBIG, step=NW)
        def _(u):
            f = u // NBIG
            v0 = pl.multiple_of((u % NBIG) * C, C)
            pltpu.sync_copy(embT_hbm.at[f, :, pl.ds(v0, C)], buf_v)
            transpose_rows(C)
            r0 = pl.multiple_of(v0 * D // 128, C * D // 128)
            pltpu.sync_copy(tbuf_v, out_hbm.at[f, pl.ds(r0, C * D // 128)])

        if C2 > 0:
            @pl.loop(wid, F, step=NW)
            def _(f):
                pltpu.sync_copy(embT_hbm.at[f, :, pl.ds(VT, C2)],
                                buf_v.at[:, pl.ds(0, C2)])
                transpose_rows(C2)
                pltpu.sync_copy(tbuf_v.at[pl.ds(0, C2 * D // 128)],
                                out_hbm.at[f, pl.ds(VT * D // 128, C2 * D // 128)])

        if V > VT2:
            @pl.loop(wid, F, step=NW)
            def _(f):
                pltpu.sync_copy(tail_hbm.at[f],
                                out_hbm.at[f, pl.ds(VT2 * D // 128, TROWS)])

    return k(embT, tail_slabs)


def _sc_gather(emb8, fm_flat, xcT, D):
    F, nslab, slab_w = emb8.shape
    per_slab = slab_w // D
    V = nslab * per_slab
    B = xcT.shape[1]
    mesh = plsc.VectorSubcoreMesh(core_axis_name="core", subcore_axis_name="subcore")
    info = plsc.get_sparse_core_info()
    NW = info.num_cores * info.num_subcores
    S = B // NW
    L = info.num_lanes

    @functools.partial(
        pl.kernel,
        out_type=[
            jax.ShapeDtypeStruct((F, D, B), jnp.float32),
            jax.ShapeDtypeStruct((B,), jnp.float32),
        ],
        mesh=mesh,
        scratch_types=[
            pltpu.VMEM((S,), jnp.int32),
            pltpu.VMEM((S,), jnp.int32),
            pltpu.VMEM((S, slab_w), jnp.float32),
            pltpu.VMEM((D, S), jnp.float32),
            pltpu.VMEM((S,), jnp.float32),
            pltpu.VMEM((S,), jnp.float32),
            pltpu.SemaphoreType.DMA,
        ],
        compiler_params=_sc_compiler_params(),
    )
    def k(emb_hbm, fm_hbm, xc_hbm, oemb_hbm, ofm_hbm,
          idx_v, sidx_v, slab_v, rowsT_v, fmv_v, fmacc_v, sem):
        wid = (jax.lax.axis_index("subcore") * info.num_cores
               + jax.lax.axis_index("core"))
        base = wid * S

        @pl.loop(0, S, step=L)
        def _(j):
            fmacc_v[pl.ds(j, L)] = jnp.zeros((L,), jnp.float32)

        @pl.loop(0, F)
        def _(f):
            pltpu.sync_copy(xc_hbm.at[f, pl.ds(base, S)], idx_v)

            @pl.loop(0, S, step=L)
            def _(j):
                sidx_v[pl.ds(j, L)] = jax.lax.shift_right_logical(
                    idx_v[pl.ds(j, L)], 3)

            pltpu.async_copy(emb_hbm.at[f].at[sidx_v], slab_v, sem).wait()

            @pl.loop(0, S, step=L)
            def _(i):
                lanes = jax.lax.iota(jnp.int32, L)
                rows = lanes + i
                colb = (idx_v[pl.ds(i, L)] & (per_slab - 1)) * D
                for d in range(D):
                    vals = plsc.load_gather(slab_v, [rows, colb + d])
                    rowsT_v[d, pl.ds(i, L)] = vals

            pltpu.sync_copy(rowsT_v, oemb_hbm.at[f, :, pl.ds(base, S)])

            off = f * V

            @pl.loop(0, S, step=L)
            def _(j):
                sidx_v[pl.ds(j, L)] = idx_v[pl.ds(j, L)] + off

            pltpu.async_copy(fm_hbm.at[sidx_v], fmv_v, sem).wait()

            @pl.loop(0, S, step=L)
            def _(j):
                fmacc_v[pl.ds(j, L)] = fmacc_v[pl.ds(j, L)] + fmv_v[pl.ds(j, L)]

        pltpu.sync_copy(fmacc_v, ofm_hbm.at[pl.ds(base, S)])

    return k(emb8, fm_flat, xcT)


def _tc1_body(emb_ref, xn_ref, fm1_ref, w1a_ref, w1b_ref, b1_ref, b3_ref,
              h1_ref, fmsum_ref, s_ref, ss_ref, *, nf):
    h = jax.lax.dot(w1b_ref[...], xn_ref[...],
                    precision=jax.lax.Precision.HIGHEST)
    h = h + b1_ref[...]
    s16 = None
    sq = None
    for f in range(nf):
        e = emb_ref[f]
        s16 = e if s16 is None else s16 + e
        esq = jnp.sum(e * e, axis=0)
        sq = esq if sq is None else sq + esq
        d = e.shape[0]
        h = h + jnp.dot(w1a_ref[:, pl.ds(f * d, d)], e.astype(jnp.bfloat16),
                        preferred_element_type=jnp.float32)
    fm2 = 0.5 * (jnp.sum(s16 * s16, axis=0) - sq)
    fmsum_ref[...] = (fm1_ref[0, :] + fm2 + b3_ref[0, 0])[None, :]
    h1_ref[...] = h

    @pl.when(pl.program_id(0) == 0)
    def _():
        s_ref[...] = jnp.zeros_like(s_ref)
        ss_ref[...] = jnp.zeros_like(ss_ref)

    s_ref[...] += jnp.sum(h, axis=1, keepdims=True)
    ss_ref[...] += jnp.sum(h * h, axis=1, keepdims=True)


def _tc2_body(h1_ref, s_ref, ss_ref, g1_ref, be1_ref, w2_ref, b2_ref,
              h2_ref, s2_ref, ss2_ref, *, batch):
    mean = s_ref[...] * (1.0 / batch)
    var = ss_ref[...] * (1.0 / batch) - mean * mean
    inv = g1_ref[...] / jnp.sqrt(var + EPS)
    a = jnp.maximum(h1_ref[...] * inv + (be1_ref[...] - mean * inv), 0.0)
    h = jnp.dot(w2_ref[...], a.astype(jnp.bfloat16),
                preferred_element_type=jnp.float32)
    h = h + b2_ref[...]
    h2_ref[...] = h

    @pl.when(pl.program_id(0) == 0)
    def _():
        s2_ref[...] = jnp.zeros_like(s2_ref)
        ss2_ref[...] = jnp.zeros_like(ss2_ref)

    s2_ref[...] += jnp.sum(h, axis=1, keepdims=True)
    ss2_ref[...] += jnp.sum(h * h, axis=1, keepdims=True)


def _tc3_body(h2_ref, s2_ref, ss2_ref, g2_ref, be2_ref, w3_ref, fmsum_ref,
              out_ref, *, batch):
    mean = s2_ref[...] * (1.0 / batch)
    var = ss2_ref[...] * (1.0 / batch) - mean * mean
    inv = g2_ref[...] / jnp.sqrt(var + EPS)
    a = jnp.maximum(h2_ref[...] * inv + (be2_ref[...] - mean * inv), 0.0)
    dnn = jnp.sum(a * w3_ref[...], axis=0)
    logit = dnn + fmsum_ref[0, :]
    out_ref[...] = jax.nn.sigmoid(logit)[None, :]


def kernel(x_cat, x_num, emb_tables, fm_table, offsets,
           W1, b1, g1, be1, W2, b2, g2, be2, W3, b3):
    B, F = x_cat.shape
    _, V, D = emb_tables.shape
    NUM = x_num.shape[1]
    H = W1.shape[1]
    NB = B // TB

    xcT = jnp.transpose(x_cat).astype(jnp.int32)
    embT = jnp.transpose(emb_tables, (0, 2, 1))

    cutoff = (V // 128) * 128
    tail_slabs = jnp.transpose(embT[:, :, cutoff:], (0, 2, 1)).reshape(
        F, (V - cutoff) * D // (8 * D), 8 * D)
    emb8 = _sc_repack(embT, tail_slabs)
    emb3, fm_first = _sc_gather(emb8, fm_table.reshape(F * V), xcT, D)
    fm1r = fm_first.reshape(1, B)

    xnT = jnp.transpose(x_num)
    w1aT = jnp.transpose(W1[:F * D]).astype(jnp.bfloat16)
    w1bT = jnp.transpose(W1[F * D:])
    w2T = jnp.transpose(W2).astype(jnp.bfloat16)
    b1c = b1.reshape(H, 1)
    b2c = b2.reshape(H, 1)
    g1c = g1.reshape(H, 1)
    be1c = be1.reshape(H, 1)
    g2c = g2.reshape(H, 1)
    be2c = be2.reshape(H, 1)
    w3c = W3.reshape(H, 1)
    b3r = b3.reshape(1, 1)

    const = lambda shape: pl.BlockSpec(shape, lambda i: (0, 0))
    col = lambda shape: pl.BlockSpec(shape, lambda i: (0, i))

    f32 = jnp.float32
    h1, fmsum, s1, ss1 = pl.pallas_call(
        functools.partial(_tc1_body, nf=F),
        grid=(NB,),
        in_specs=[
            pl.BlockSpec((F, D, TB), lambda i: (0, 0, i)),
            col((NUM, TB)), col((1, TB)),
            const((H, F * D)), const((H, NUM)),
            const((H, 1)), const((1, 1)),
        ],
        out_specs=[col((H, TB)), col((1, TB)), const((H, 1)), const((H, 1))],
        out_shape=[
            jax.ShapeDtypeStruct((H, B), f32),
            jax.ShapeDtypeStruct((1, B), f32),
            jax.ShapeDtypeStruct((H, 1), f32),
            jax.ShapeDtypeStruct((H, 1), f32),
        ],
    )(emb3, xnT, fm1r, w1aT, w1bT, b1c, b3r)

    h2, s2, ss2 = pl.pallas_call(
        functools.partial(_tc2_body, batch=B),
        grid=(NB,),
        in_specs=[
            col((H, TB)), const((H, 1)), const((H, 1)),
            const((H, 1)), const((H, 1)), const((H, H)), const((H, 1)),
        ],
        out_specs=[col((H, TB)), const((H, 1)), const((H, 1))],
        out_shape=[
            jax.ShapeDtypeStruct((H, B), f32),
            jax.ShapeDtypeStruct((H, 1), f32),
            jax.ShapeDtypeStruct((H, 1), f32),
        ],
    )(h1, s1, ss1, g1c, be1c, w2T, b2c)

    out2d = pl.pallas_call(
        functools.partial(_tc3_body, batch=B),
        grid=(NB,),
        in_specs=[
            col((H, TB)), const((H, 1)), const((H, 1)),
            const((H, 1)), const((H, 1)), const((H, 1)), col((1, TB)),
        ],
        out_specs=col((1, TB)),
        out_shape=jax.ShapeDtypeStruct((1, B), f32),
    )(h2, s2, ss2, g2c, be2c, w3c, fmsum)

    return out2d.reshape(B)


# speedup vs baseline: 2.0549x; 1.1596x over previous
import dataclasses
import functools

import jax
import jax.numpy as jnp
from jax.experimental import pallas as pl
from jax.experimental.pallas import tpu as pltpu
from jax.experimental.pallas import tpu_sc as plsc

EPS = 1e-5
TB = 512


def _sc_compiler_params():
    cp = pltpu.CompilerParams(use_tc_tiling_on_sc=True)
    if "needs_layout_passes" in pltpu.CompilerParams.__dataclass_fields__:
        cp = dataclasses.replace(cp, needs_layout_passes=False)
    return cp


def _sc_repack(embT, tail_slabs):
    F, D, V = embT.shape
    C = 1920
    mesh = plsc.VectorSubcoreMesh(core_axis_name="core", subcore_axis_name="subcore")
    info = plsc.get_sparse_core_info()
    NW = info.num_cores * info.num_subcores
    L = info.num_lanes

    NBIG = V // C
    VT = NBIG * C
    C2 = ((V - VT) // 128) * 128
    VT2 = VT + C2
    TROWS = (V - VT2) * D // 128

    CR = C * D // 128
    NU = F * NBIG
    KMAX = ((NU + NW - 1) // NW + 1) // 2 * 2

    @functools.partial(
        pl.kernel,
        out_type=jax.ShapeDtypeStruct((F, V // 8, 8 * D), jnp.float32),
        mesh=mesh,
        scratch_types=[
            pltpu.VMEM((D, C), jnp.float32),
            pltpu.VMEM((D, C), jnp.float32),
            pltpu.VMEM((CR, 128), jnp.float32),
            pltpu.VMEM((CR, 128), jnp.float32),
            pltpu.SemaphoreType.DMA,
            pltpu.SemaphoreType.DMA,
            pltpu.SemaphoreType.DMA,
            pltpu.SemaphoreType.DMA,
        ],
        compiler_params=_sc_compiler_params(),
    )
    def k(embT_hbm, tail_hbm, out_hbm, buf0_v, buf1_v, tbuf0_v, tbuf1_v,
          rs0, rs1, ws0, ws1):
        wid = (jax.lax.axis_index("subcore") * info.num_cores
               + jax.lax.axis_index("core"))
        bufs, tbufs = (buf0_v, buf1_v), (tbuf0_v, tbuf1_v)
        rsems, wsems = (rs0, rs1), (ws0, ws1)

        def unit_src(u):
            f = u // NBIG
            v0 = pl.multiple_of((u % NBIG) * C, C)
            return embT_hbm.at[f, :, pl.ds(v0, C)]

        def unit_dst(u):
            f = u // NBIG
            r0 = pl.multiple_of((u % NBIG) * CR, CR)
            return out_hbm.at[f, pl.ds(r0, CR)]

        def transpose_rows(nv, src, dst):
            @pl.loop(0, nv, step=L)
            def _(v):
                vv = jax.lax.iota(jnp.int32, L) + v
                rowv = jax.lax.shift_right_logical(vv, 3)
                colb = (vv & 7) * D
                for d in range(D):
                    vals = src[d, pl.ds(v, L)]
                    plsc.store_scatter(dst, [rowv, colb + d], vals)

        for b in (0, 1):
            u = wid + b * NW

            @pl.when(u < NU)
            def _(u=u, b=b):
                pltpu.async_copy(unit_src(u), bufs[b], rsems[b])

        @pl.loop(0, KMAX, step=2)
        def _(kk):
            for b in (0, 1):
                j = kk + b
                u = wid + j * NW

                @pl.when(u < NU)
                def _(j=j, u=u, b=b):
                    pltpu.make_async_copy(unit_src(u), bufs[b], rsems[b]).wait()

                    @pl.when(j >= 2)
                    def _():
                        pltpu.make_async_copy(tbufs[b], unit_dst(u - 2 * NW),
                                              wsems[b]).wait()

                    transpose_rows(C, bufs[b], tbufs[b])
                    pltpu.async_copy(tbufs[b], unit_dst(u), wsems[b])

                    @pl.when(u + 2 * NW < NU)
                    def _():
                        pltpu.async_copy(unit_src(u + 2 * NW), bufs[b], rsems[b])

        jmax = (NU - 1 - wid) // NW
        for b in (0, 1):
            jb = jmax - ((jmax - b) % 2)

            @pl.when(jb >= 0)
            def _(jb=jb, b=b):
                pltpu.make_async_copy(tbufs[b], unit_dst(wid + jb * NW),
                                      wsems[b]).wait()

        if C2 > 0:
            @pl.loop(wid, F, step=NW)
            def _(f):
                pltpu.sync_copy(embT_hbm.at[f, :, pl.ds(VT, C2)],
                                buf0_v.at[:, pl.ds(0, C2)])
                transpose_rows(C2, buf0_v, tbuf0_v)
                pltpu.sync_copy(tbuf0_v.at[pl.ds(0, C2 * D // 128)],
                                out_hbm.at[f, pl.ds(VT * D // 128, C2 * D // 128)])

        if V > VT2:
            @pl.loop(wid, F, step=NW)
            def _(f):
                pltpu.sync_copy(tail_hbm.at[f],
                                out_hbm.at[f, pl.ds(VT2 * D // 128, TROWS)])

    return k(embT, tail_slabs)


def _sc_gather(emb8, fm_flat, xcT, D):
    F, nslab, slab_w = emb8.shape
    per_slab = slab_w // D
    V = nslab * per_slab
    B = xcT.shape[1]
    mesh = plsc.VectorSubcoreMesh(core_axis_name="core", subcore_axis_name="subcore")
    info = plsc.get_sparse_core_info()
    NW = info.num_cores * info.num_subcores
    S = B // NW
    L = info.num_lanes

    @functools.partial(
        pl.kernel,
        out_type=[
            jax.ShapeDtypeStruct((F, D, B), jnp.float32),
            jax.ShapeDtypeStruct((B,), jnp.float32),
        ],
        mesh=mesh,
        scratch_types=[
            pltpu.VMEM((S,), jnp.int32),
            pltpu.VMEM((S,), jnp.int32),
            pltpu.VMEM((S, slab_w), jnp.float32),
            pltpu.VMEM((D, S), jnp.float32),
            pltpu.VMEM((S,), jnp.float32),
            pltpu.VMEM((S,), jnp.float32),
            pltpu.SemaphoreType.DMA,
        ],
        compiler_params=_sc_compiler_params(),
    )
    def k(emb_hbm, fm_hbm, xc_hbm, oemb_hbm, ofm_hbm,
          idx_v, sidx_v, slab_v, rowsT_v, fmv_v, fmacc_v, sem):
        wid = (jax.lax.axis_index("subcore") * info.num_cores
               + jax.lax.axis_index("core"))
        base = wid * S

        @pl.loop(0, S, step=L)
        def _(j):
            fmacc_v[pl.ds(j, L)] = jnp.zeros((L,), jnp.float32)

        @pl.loop(0, F)
        def _(f):
            pltpu.sync_copy(xc_hbm.at[f, pl.ds(base, S)], idx_v)

            @pl.loop(0, S, step=L)
            def _(j):
                sidx_v[pl.ds(j, L)] = jax.lax.shift_right_logical(
                    idx_v[pl.ds(j, L)], 3)

            pltpu.async_copy(emb_hbm.at[f].at[sidx_v], slab_v, sem).wait()

            @pl.loop(0, S, step=L)
            def _(i):
                lanes = jax.lax.iota(jnp.int32, L)
                rows = lanes + i
                colb = (idx_v[pl.ds(i, L)] & (per_slab - 1)) * D
                for d in range(D):
                    vals = plsc.load_gather(slab_v, [rows, colb + d])
                    rowsT_v[d, pl.ds(i, L)] = vals

            pltpu.sync_copy(rowsT_v, oemb_hbm.at[f, :, pl.ds(base, S)])

            off = f * V

            @pl.loop(0, S, step=L)
            def _(j):
                sidx_v[pl.ds(j, L)] = idx_v[pl.ds(j, L)] + off

            pltpu.async_copy(fm_hbm.at[sidx_v], fmv_v, sem).wait()

            @pl.loop(0, S, step=L)
            def _(j):
                fmacc_v[pl.ds(j, L)] = fmacc_v[pl.ds(j, L)] + fmv_v[pl.ds(j, L)]

        pltpu.sync_copy(fmacc_v, ofm_hbm.at[pl.ds(base, S)])

    return k(emb8, fm_flat, xcT)


def _tc1_body(emb_ref, xn_ref, fm1_ref, w1a_ref, w1b_ref, b1_ref, b3_ref,
              h1_ref, fmsum_ref, s_ref, ss_ref, *, nf):
    h = jax.lax.dot(w1b_ref[...], xn_ref[...],
                    precision=jax.lax.Precision.HIGHEST)
    h = h + b1_ref[...]
    s16 = None
    sq = None
    for f in range(nf):
        e = emb_ref[f]
        s16 = e if s16 is None else s16 + e
        esq = jnp.sum(e * e, axis=0)
        sq = esq if sq is None else sq + esq
        d = e.shape[0]
        h = h + jnp.dot(w1a_ref[:, pl.ds(f * d, d)], e.astype(jnp.bfloat16),
                        preferred_element_type=jnp.float32)
    fm2 = 0.5 * (jnp.sum(s16 * s16, axis=0) - sq)
    fmsum_ref[...] = (fm1_ref[0, :] + fm2 + b3_ref[0, 0])[None, :]
    h1_ref[...] = h

    @pl.when(pl.program_id(0) == 0)
    def _():
        s_ref[...] = jnp.zeros_like(s_ref)
        ss_ref[...] = jnp.zeros_like(ss_ref)

    s_ref[...] += jnp.sum(h, axis=1, keepdims=True)
    ss_ref[...] += jnp.sum(h * h, axis=1, keepdims=True)


def _tc2_body(h1_ref, s_ref, ss_ref, g1_ref, be1_ref, w2_ref, b2_ref,
              h2_ref, s2_ref, ss2_ref, *, batch):
    mean = s_ref[...] * (1.0 / batch)
    var = ss_ref[...] * (1.0 / batch) - mean * mean
    inv = g1_ref[...] / jnp.sqrt(var + EPS)
    a = jnp.maximum(h1_ref[...] * inv + (be1_ref[...] - mean * inv), 0.0)
    h = jnp.dot(w2_ref[...], a.astype(jnp.bfloat16),
                preferred_element_type=jnp.float32)
    h = h + b2_ref[...]
    h2_ref[...] = h

    @pl.when(pl.program_id(0) == 0)
    def _():
        s2_ref[...] = jnp.zeros_like(s2_ref)
        ss2_ref[...] = jnp.zeros_like(ss2_ref)

    s2_ref[...] += jnp.sum(h, axis=1, keepdims=True)
    ss2_ref[...] += jnp.sum(h * h, axis=1, keepdims=True)


def _tc3_body(h2_ref, s2_ref, ss2_ref, g2_ref, be2_ref, w3_ref, fmsum_ref,
              out_ref, *, batch):
    mean = s2_ref[...] * (1.0 / batch)
    var = ss2_ref[...] * (1.0 / batch) - mean * mean
    inv = g2_ref[...] / jnp.sqrt(var + EPS)
    a = jnp.maximum(h2_ref[...] * inv + (be2_ref[...] - mean * inv), 0.0)
    dnn = jnp.sum(a * w3_ref[...], axis=0)
    logit = dnn + fmsum_ref[0, :]
    out_ref[...] = jax.nn.sigmoid(logit)[None, :]


def kernel(x_cat, x_num, emb_tables, fm_table, offsets,
           W1, b1, g1, be1, W2, b2, g2, be2, W3, b3):
    B, F = x_cat.shape
    _, V, D = emb_tables.shape
    NUM = x_num.shape[1]
    H = W1.shape[1]
    NB = B // TB

    xcT = jnp.transpose(x_cat).astype(jnp.int32)
    embT = jnp.transpose(emb_tables, (0, 2, 1))

    cutoff = (V // 128) * 128
    tail_slabs = jnp.transpose(embT[:, :, cutoff:], (0, 2, 1)).reshape(
        F, (V - cutoff) * D // (8 * D), 8 * D)
    emb8 = _sc_repack(embT, tail_slabs)
    emb3, fm_first = _sc_gather(emb8, fm_table.reshape(F * V), xcT, D)
    fm1r = fm_first.reshape(1, B)

    xnT = jnp.transpose(x_num)
    w1aT = jnp.transpose(W1[:F * D]).astype(jnp.bfloat16)
    w1bT = jnp.transpose(W1[F * D:])
    w2T = jnp.transpose(W2).astype(jnp.bfloat16)
    b1c = b1.reshape(H, 1)
    b2c = b2.reshape(H, 1)
    g1c = g1.reshape(H, 1)
    be1c = be1.reshape(H, 1)
    g2c = g2.reshape(H, 1)
    be2c = be2.reshape(H, 1)
    w3c = W3.reshape(H, 1)
    b3r = b3.reshape(1, 1)

    const = lambda shape: pl.BlockSpec(shape, lambda i: (0, 0))
    col = lambda shape: pl.BlockSpec(shape, lambda i: (0, i))

    f32 = jnp.float32
    h1, fmsum, s1, ss1 = pl.pallas_call(
        functools.partial(_tc1_body, nf=F),
        grid=(NB,),
        in_specs=[
            pl.BlockSpec((F, D, TB), lambda i: (0, 0, i)),
            col((NUM, TB)), col((1, TB)),
            const((H, F * D)), const((H, NUM)),
            const((H, 1)), const((1, 1)),
        ],
        out_specs=[col((H, TB)), col((1, TB)), const((H, 1)), const((H, 1))],
        out_shape=[
            jax.ShapeDtypeStruct((H, B), f32),
            jax.ShapeDtypeStruct((1, B), f32),
            jax.ShapeDtypeStruct((H, 1), f32),
            jax.ShapeDtypeStruct((H, 1), f32),
        ],
    )(emb3, xnT, fm1r, w1aT, w1bT, b1c, b3r)

    h2, s2, ss2 = pl.pallas_call(
        functools.partial(_tc2_body, batch=B),
        grid=(NB,),
        in_specs=[
            col((H, TB)), const((H, 1)), const((H, 1)),
            const((H, 1)), const((H, 1)), const((H, H)), const((H, 1)),
        ],
        out_specs=[col((H, TB)), const((H, 1)), const((H, 1))],
        out_shape=[
            jax.ShapeDtypeStruct((H, B), f32),
            jax.ShapeDtypeStruct((H, 1), f32),
            jax.ShapeDtypeStruct((H, 1), f32),
        ],
    )(h1, s1, ss1, g1c, be1c, w2T, b2c)

    out2d = pl.pallas_call(
        functools.partial(_tc3_body, batch=B),
        grid=(NB,),
        in_specs=[
            col((H, TB)), const((H, 1)), const((H, 1)),
            const((H, 1)), const((H, 1)), const((H, 1)), col((1, TB)),
        ],
        out_specs=col((1, TB)),
        out_shape=jax.ShapeDtypeStruct((1, B), f32),
    )(h2, s2, ss2, g2c, be2c, w3c, fmsum)

    return out2d.reshape(B)


# speedup vs baseline: 2.4243x; 1.1798x over previous
import dataclasses
import functools

import jax
import jax.numpy as jnp
from jax.experimental import pallas as pl
from jax.experimental.pallas import tpu as pltpu
from jax.experimental.pallas import tpu_sc as plsc

EPS = 1e-5
TB = 512


def _sc_compiler_params():
    cp = pltpu.CompilerParams(use_tc_tiling_on_sc=True)
    if "needs_layout_passes" in pltpu.CompilerParams.__dataclass_fields__:
        cp = dataclasses.replace(cp, needs_layout_passes=False)
    return cp


def _sc_repack(embT, tail_slabs):
    F, D, V = embT.shape
    C = 1920
    mesh = plsc.VectorSubcoreMesh(core_axis_name="core", subcore_axis_name="subcore")
    info = plsc.get_sparse_core_info()
    NW = info.num_cores * info.num_subcores
    L = info.num_lanes

    NBIG = V // C
    VT = NBIG * C
    C2 = ((V - VT) // 128) * 128
    VT2 = VT + C2
    TROWS = (V - VT2) * D // 128

    CR = C * D // 128
    NU = F * NBIG
    KMAX = ((NU + NW - 1) // NW + 1) // 2 * 2

    @functools.partial(
        pl.kernel,
        out_type=jax.ShapeDtypeStruct((F, V // 8, 8 * D), jnp.float32),
        mesh=mesh,
        scratch_types=[
            pltpu.VMEM((D, C), jnp.float32),
            pltpu.VMEM((D, C), jnp.float32),
            pltpu.VMEM((CR, 128), jnp.float32),
            pltpu.VMEM((CR, 128), jnp.float32),
            pltpu.SemaphoreType.DMA,
            pltpu.SemaphoreType.DMA,
            pltpu.SemaphoreType.DMA,
            pltpu.SemaphoreType.DMA,
        ],
        compiler_params=_sc_compiler_params(),
    )
    def k(embT_hbm, tail_hbm, out_hbm, buf0_v, buf1_v, tbuf0_v, tbuf1_v,
          rs0, rs1, ws0, ws1):
        wid = (jax.lax.axis_index("subcore") * info.num_cores
               + jax.lax.axis_index("core"))
        bufs, tbufs = (buf0_v, buf1_v), (tbuf0_v, tbuf1_v)
        rsems, wsems = (rs0, rs1), (ws0, ws1)

        def unit_src(u):
            f = u // NBIG
            v0 = pl.multiple_of((u % NBIG) * C, C)
            return embT_hbm.at[f, :, pl.ds(v0, C)]

        def unit_dst(u):
            f = u // NBIG
            r0 = pl.multiple_of((u % NBIG) * CR, CR)
            return out_hbm.at[f, pl.ds(r0, CR)]

        def transpose_rows(nv, src, dst):
            @pl.loop(0, nv, step=L)
            def _(v):
                vv = jax.lax.iota(jnp.int32, L) + v
                rowv = jax.lax.shift_right_logical(vv, 3)
                colb = (vv & 7) * D
                for d in range(D):
                    vals = src[d, pl.ds(v, L)]
                    plsc.store_scatter(dst, [rowv, colb + d], vals)

        for b in (0, 1):
            u = wid + b * NW

            @pl.when(u < NU)
            def _(u=u, b=b):
                pltpu.async_copy(unit_src(u), bufs[b], rsems[b])

        @pl.loop(0, KMAX, step=2)
        def _(kk):
            for b in (0, 1):
                j = kk + b
                u = wid + j * NW

                @pl.when(u < NU)
                def _(j=j, u=u, b=b):
                    pltpu.make_async_copy(unit_src(u), bufs[b], rsems[b]).wait()

                    @pl.when(j >= 2)
                    def _():
                        pltpu.make_async_copy(tbufs[b], unit_dst(u - 2 * NW),
                                              wsems[b]).wait()

                    transpose_rows(C, bufs[b], tbufs[b])
                    pltpu.async_copy(tbufs[b], unit_dst(u), wsems[b])

                    @pl.when(u + 2 * NW < NU)
                    def _():
                        pltpu.async_copy(unit_src(u + 2 * NW), bufs[b], rsems[b])

        jmax = (NU - 1 - wid) // NW
        for b in (0, 1):
            jb = jmax - ((jmax - b) % 2)

            @pl.when(jb >= 0)
            def _(jb=jb, b=b):
                pltpu.make_async_copy(tbufs[b], unit_dst(wid + jb * NW),
                                      wsems[b]).wait()

        if C2 > 0:
            @pl.loop(wid, F, step=NW)
            def _(f):
                pltpu.sync_copy(embT_hbm.at[f, :, pl.ds(VT, C2)],
                                buf0_v.at[:, pl.ds(0, C2)])
                transpose_rows(C2, buf0_v, tbuf0_v)
                pltpu.sync_copy(tbuf0_v.at[pl.ds(0, C2 * D // 128)],
                                out_hbm.at[f, pl.ds(VT * D // 128, C2 * D // 128)])

        if V > VT2:
            @pl.loop(wid, F, step=NW)
            def _(f):
                pltpu.sync_copy(tail_hbm.at[f],
                                out_hbm.at[f, pl.ds(VT2 * D // 128, TROWS)])

    return k(embT, tail_slabs)


def _sc_gather(emb8, fm_flat, xcT, D):
    F, nslab, slab_w = emb8.shape
    per_slab = slab_w // D
    V = nslab * per_slab
    B = xcT.shape[1]
    mesh = plsc.VectorSubcoreMesh(core_axis_name="core", subcore_axis_name="subcore")
    info = plsc.get_sparse_core_info()
    NW = info.num_cores * info.num_subcores
    S = B // NW
    L = info.num_lanes

    NH = 2
    SH = S // NH
    NUG = F * NH

    @functools.partial(
        pl.kernel,
        out_type=[
            jax.ShapeDtypeStruct((F, D, B), jnp.float32),
            jax.ShapeDtypeStruct((B,), jnp.float32),
        ],
        mesh=mesh,
        scratch_types=[
            pltpu.VMEM((SH,), jnp.int32),
            pltpu.VMEM((SH,), jnp.int32),
            pltpu.VMEM((SH,), jnp.int32),
            pltpu.VMEM((SH,), jnp.int32),
            pltpu.VMEM((SH, slab_w), jnp.float32),
            pltpu.VMEM((SH, slab_w), jnp.float32),
            pltpu.VMEM((D, SH), jnp.float32),
            pltpu.VMEM((D, SH), jnp.float32),
            pltpu.VMEM((SH,), jnp.float32),
            pltpu.VMEM((SH,), jnp.float32),
            pltpu.VMEM((S,), jnp.float32),
            pltpu.SemaphoreType.DMA,
            pltpu.SemaphoreType.DMA,
            pltpu.SemaphoreType.DMA,
            pltpu.SemaphoreType.DMA,
            pltpu.SemaphoreType.DMA,
            pltpu.SemaphoreType.DMA,
        ],
        compiler_params=_sc_compiler_params(),
    )
    def k(emb_hbm, fm_hbm, xc_hbm, oemb_hbm, ofm_hbm,
          idx0, idx1, sidx0, sidx1, slab0, slab1, rt0, rt1, fmv0, fmv1,
          fmacc_v, rs0, rs1, ws0, ws1, fs0, fs1):
        wid = (jax.lax.axis_index("subcore") * info.num_cores
               + jax.lax.axis_index("core"))
        base = wid * S
        idxs, sidxs = (idx0, idx1), (sidx0, sidx1)
        slabs, rts, fmvs = (slab0, slab1), (rt0, rt1), (fmv0, fmv1)
        rsems, wsems, fsems = (rs0, rs1), (ws0, ws1), (fs0, fs1)

        @pl.loop(0, S, step=L)
        def _(j):
            fmacc_v[pl.ds(j, L)] = jnp.zeros((L,), jnp.float32)

        def stage_a(u, b):
            f = u // NH
            ub = base + (u % NH) * SH
            pltpu.sync_copy(xc_hbm.at[f, pl.ds(ub, SH)], idxs[b])

            @pl.loop(0, SH, step=L)
            def _(j):
                sidxs[b][pl.ds(j, L)] = jax.lax.shift_right_logical(
                    idxs[b][pl.ds(j, L)], 3)

            pltpu.async_copy(emb_hbm.at[f].at[sidxs[b]], slabs[b], rsems[b])

        def out_dst(u):
            f = u // NH
            ub = base + (u % NH) * SH
            return oemb_hbm.at[f, :, pl.ds(ub, SH)]

        for b in (0, 1):
            stage_a(b, b)

        @pl.loop(0, NUG, step=2)
        def _(kk):
            for b in (0, 1):
                u = kk + b
                f = u // NH
                pltpu.make_async_copy(emb_hbm.at[f].at[sidxs[b]], slabs[b],
                                      rsems[b]).wait()

                @pl.when(u >= 2)
                def _(u=u, b=b):
                    pltpu.make_async_copy(rts[b], out_dst(u - 2), wsems[b]).wait()

                off = f * V

                @pl.loop(0, SH, step=L)
                def _(j):
                    sidxs[b][pl.ds(j, L)] = idxs[b][pl.ds(j, L)] + off

                pltpu.async_copy(fm_hbm.at[sidxs[b]], fmvs[b], fsems[b])

                @pl.loop(0, SH, step=L)
                def _(i):
                    lanes = jax.lax.iota(jnp.int32, L)
                    rows = lanes + i
                    colb = (idxs[b][pl.ds(i, L)] & (per_slab - 1)) * D
                    for d in range(D):
                        vals = plsc.load_gather(slabs[b], [rows, colb + d])
                        rts[b][d, pl.ds(i, L)] = vals

                pltpu.async_copy(rts[b], out_dst(u), wsems[b])

                pltpu.make_async_copy(fm_hbm.at[sidxs[b]], fmvs[b],
                                      fsems[b]).wait()
                ab = (u % NH) * SH

                @pl.loop(0, SH, step=L)
                def _(j):
                    fmacc_v[pl.ds(ab + j, L)] = (fmacc_v[pl.ds(ab + j, L)]
                                                 + fmvs[b][pl.ds(j, L)])

                @pl.when(u + 2 < NUG)
                def _(u=u, b=b):
                    stage_a(u + 2, b)

        for b in (0, 1):
            pltpu.make_async_copy(rts[b], out_dst(NUG - 2 + b), wsems[b]).wait()

        pltpu.sync_copy(fmacc_v, ofm_hbm.at[pl.ds(base, S)])

    return k(emb8, fm_flat, xcT)


def _tc1_body(emb_ref, xn_ref, fm1_ref, w1a_ref, w1b_ref, b1_ref, b3_ref,
              h1_ref, fmsum_ref, s_ref, ss_ref, *, nf):
    h = jax.lax.dot(w1b_ref[...], xn_ref[...],
                    precision=jax.lax.Precision.HIGHEST)
    h = h + b1_ref[...]
    s16 = None
    sq = None
    for f in range(nf):
        e = emb_ref[f]
        s16 = e if s16 is None else s16 + e
        esq = jnp.sum(e * e, axis=0)
        sq = esq if sq is None else sq + esq
        d = e.shape[0]
        h = h + jnp.dot(w1a_ref[:, pl.ds(f * d, d)], e.astype(jnp.bfloat16),
                        preferred_element_type=jnp.float32)
    fm2 = 0.5 * (jnp.sum(s16 * s16, axis=0) - sq)
    fmsum_ref[...] = (fm1_ref[0, :] + fm2 + b3_ref[0, 0])[None, :]
    h1_ref[...] = h

    @pl.when(pl.program_id(0) == 0)
    def _():
        s_ref[...] = jnp.zeros_like(s_ref)
        ss_ref[...] = jnp.zeros_like(ss_ref)

    s_ref[...] += jnp.sum(h, axis=1, keepdims=True)
    ss_ref[...] += jnp.sum(h * h, axis=1, keepdims=True)


def _tc2_body(h1_ref, s_ref, ss_ref, g1_ref, be1_ref, w2_ref, b2_ref,
              h2_ref, s2_ref, ss2_ref, *, batch):
    mean = s_ref[...] * (1.0 / batch)
    var = ss_ref[...] * (1.0 / batch) - mean * mean
    inv = g1_ref[...] / jnp.sqrt(var + EPS)
    a = jnp.maximum(h1_ref[...] * inv + (be1_ref[...] - mean * inv), 0.0)
    h = jnp.dot(w2_ref[...], a.astype(jnp.bfloat16),
                preferred_element_type=jnp.float32)
    h = h + b2_ref[...]
    h2_ref[...] = h

    @pl.when(pl.program_id(0) == 0)
    def _():
        s2_ref[...] = jnp.zeros_like(s2_ref)
        ss2_ref[...] = jnp.zeros_like(ss2_ref)

    s2_ref[...] += jnp.sum(h, axis=1, keepdims=True)
    ss2_ref[...] += jnp.sum(h * h, axis=1, keepdims=True)


def _tc3_body(h2_ref, s2_ref, ss2_ref, g2_ref, be2_ref, w3_ref, fmsum_ref,
              out_ref, *, batch):
    mean = s2_ref[...] * (1.0 / batch)
    var = ss2_ref[...] * (1.0 / batch) - mean * mean
    inv = g2_ref[...] / jnp.sqrt(var + EPS)
    a = jnp.maximum(h2_ref[...] * inv + (be2_ref[...] - mean * inv), 0.0)
    dnn = jnp.sum(a * w3_ref[...], axis=0)
    logit = dnn + fmsum_ref[0, :]
    out_ref[...] = jax.nn.sigmoid(logit)[None, :]


def kernel(x_cat, x_num, emb_tables, fm_table, offsets,
           W1, b1, g1, be1, W2, b2, g2, be2, W3, b3):
    B, F = x_cat.shape
    _, V, D = emb_tables.shape
    NUM = x_num.shape[1]
    H = W1.shape[1]
    NB = B // TB

    xcT = jnp.transpose(x_cat).astype(jnp.int32)
    embT = jnp.transpose(emb_tables, (0, 2, 1))

    cutoff = (V // 128) * 128
    tail_slabs = jnp.transpose(embT[:, :, cutoff:], (0, 2, 1)).reshape(
        F, (V - cutoff) * D // (8 * D), 8 * D)
    emb8 = _sc_repack(embT, tail_slabs)
    emb3, fm_first = _sc_gather(emb8, fm_table.reshape(F * V), xcT, D)
    fm1r = fm_first.reshape(1, B)

    xnT = jnp.transpose(x_num)
    w1aT = jnp.transpose(W1[:F * D]).astype(jnp.bfloat16)
    w1bT = jnp.transpose(W1[F * D:])
    w2T = jnp.transpose(W2).astype(jnp.bfloat16)
    b1c = b1.reshape(H, 1)
    b2c = b2.reshape(H, 1)
    g1c = g1.reshape(H, 1)
    be1c = be1.reshape(H, 1)
    g2c = g2.reshape(H, 1)
    be2c = be2.reshape(H, 1)
    w3c = W3.reshape(H, 1)
    b3r = b3.reshape(1, 1)

    const = lambda shape: pl.BlockSpec(shape, lambda i: (0, 0))
    col = lambda shape: pl.BlockSpec(shape, lambda i: (0, i))

    f32 = jnp.float32
    h1, fmsum, s1, ss1 = pl.pallas_call(
        functools.partial(_tc1_body, nf=F),
        grid=(NB,),
        in_specs=[
            pl.BlockSpec((F, D, TB), lambda i: (0, 0, i)),
            col((NUM, TB)), col((1, TB)),
            const((H, F * D)), const((H, NUM)),
            const((H, 1)), const((1, 1)),
        ],
        out_specs=[col((H, TB)), col((1, TB)), const((H, 1)), const((H, 1))],
        out_shape=[
            jax.ShapeDtypeStruct((H, B), f32),
            jax.ShapeDtypeStruct((1, B), f32),
            jax.ShapeDtypeStruct((H, 1), f32),
            jax.ShapeDtypeStruct((H, 1), f32),
        ],
    )(emb3, xnT, fm1r, w1aT, w1bT, b1c, b3r)

    h2, s2, ss2 = pl.pallas_call(
        functools.partial(_tc2_body, batch=B),
        grid=(NB,),
        in_specs=[
            col((H, TB)), const((H, 1)), const((H, 1)),
            const((H, 1)), const((H, 1)), const((H, H)), const((H, 1)),
        ],
        out_specs=[col((H, TB)), const((H, 1)), const((H, 1))],
        out_shape=[
            jax.ShapeDtypeStruct((H, B), f32),
            jax.ShapeDtypeStruct((H, 1), f32),
            jax.ShapeDtypeStruct((H, 1), f32),
        ],
    )(h1, s1, ss1, g1c, be1c, w2T, b2c)

    out2d = pl.pallas_call(
        functools.partial(_tc3_body, batch=B),
        grid=(NB,),
        in_specs=[
            col((H, TB)), const((H, 1)), const((H, 1)),
            const((H, 1)), const((H, 1)), const((H, 1)), col((1, TB)),
        ],
        out_specs=col((1, TB)),
        out_shape=jax.ShapeDtypeStruct((1, B), f32),
    )(h2, s2, ss2, g2c, be2c, w3c, fmsum)

    return out2d.reshape(B)
